# baseline (speedup 1.0000x reference)
import jax
import jax.numpy as jnp
from jax import lax
from jax.experimental import pallas as pl
from jax.experimental.pallas import tpu as pltpu

_CH = 256
_DEPTH = 4


def kernel(x, pos_table):
    B, S, D = x.shape
    cpb = S // _CH
    N = B * cpb

    def body(x_hbm, pos_hbm, out_hbm, pos_v, b0, b1, b2, b3,
             sp, si0, si1, si2, si3, so0, so1, so2, so3):
        bufs = (b0, b1, b2, b3)
        sin = (si0, si1, si2, si3)
        sout = (so0, so1, so2, so3)

        def src(ci):
            return x_hbm.at[ci // cpb, pl.ds((ci % cpb) * _CH, _CH)]

        def dst(ci):
            return out_hbm.at[ci // cpb, pl.ds((ci % cpb) * _CH, _CH)]

        pltpu.make_async_copy(pos_hbm, pos_v, sp).start()
        for j in range(_DEPTH - 1):
            pltpu.make_async_copy(src(j), bufs[j], sin[j]).start()
        pltpu.make_async_copy(pos_hbm, pos_v, sp).wait()

        def quad_body(g, carry):
            for b in range(_DEPTH):
                ci = _DEPTH * g + b
                pltpu.make_async_copy(src(ci), bufs[b], sin[b]).wait()
                kk = ci % cpb
                bufs[b][...] = bufs[b][...] + pos_v[pl.ds(kk * _CH, _CH), :]
                pltpu.make_async_copy(bufs[b], dst(ci), sout[b]).start()
                b3 = (b + _DEPTH - 1) % _DEPTH

                @pl.when(ci >= 1)
                def _():
                    pltpu.make_async_copy(bufs[b3], dst(ci - 1), sout[b3]).wait()

                @pl.when(ci + _DEPTH - 1 < N)
                def _():
                    pltpu.make_async_copy(
                        src(ci + _DEPTH - 1), bufs[b3], sin[b3]).start()
            return carry

        lax.fori_loop(0, N // _DEPTH, quad_body, 0)
        last_b = (N - 1) % _DEPTH
        pltpu.make_async_copy(bufs[last_b], dst(N - 1), sout[last_b]).wait()

    return pl.pallas_call(
        body,
        in_specs=[
            pl.BlockSpec(memory_space=pltpu.MemorySpace.HBM),
            pl.BlockSpec(memory_space=pltpu.MemorySpace.HBM),
        ],
        out_specs=pl.BlockSpec(memory_space=pltpu.MemorySpace.HBM),
        out_shape=jax.ShapeDtypeStruct((B, S, D), x.dtype),
        scratch_shapes=(
            [pltpu.VMEM((S, D), jnp.float32)]
            + [pltpu.VMEM((_CH, D), jnp.float32) for _ in range(_DEPTH)]
            + [pltpu.SemaphoreType.DMA] * (1 + 2 * _DEPTH)
        ),
    )(x, pos_table)


# speedup vs baseline: 1.1487x; 1.1487x over previous
import jax
import jax.numpy as jnp
from jax import lax
from jax.experimental import pallas as pl
from jax.experimental.pallas import tpu as pltpu

_CH = 512
_DEPTH = 4


def kernel(x, pos_table):
    B, S, D = x.shape
    cpb = S // _CH
    N = B * cpb

    def body(x_hbm, pos_hbm, out_hbm, pos_v, b0, b1, b2, b3,
             sp, si0, si1, si2, si3, so0, so1, so2, so3):
        bufs = (b0, b1, b2, b3)
        sin = (si0, si1, si2, si3)
        sout = (so0, so1, so2, so3)

        def src(ci):
            return x_hbm.at[ci // cpb, pl.ds((ci % cpb) * _CH, _CH)]

        def dst(ci):
            return out_hbm.at[ci // cpb, pl.ds((ci % cpb) * _CH, _CH)]

        pltpu.make_async_copy(pos_hbm, pos_v, sp).start()
        for j in range(_DEPTH - 1):
            pltpu.make_async_copy(src(j), bufs[j], sin[j]).start()
        pltpu.make_async_copy(pos_hbm, pos_v, sp).wait()

        def quad_body(g, carry):
            for b in range(_DEPTH):
                ci = _DEPTH * g + b
                pltpu.make_async_copy(src(ci), bufs[b], sin[b]).wait()
                kk = ci % cpb
                bufs[b][...] = bufs[b][...] + pos_v[pl.ds(kk * _CH, _CH), :]
                pltpu.make_async_copy(bufs[b], dst(ci), sout[b]).start()
                b3 = (b + _DEPTH - 1) % _DEPTH

                @pl.when(ci >= 1)
                def _():
                    pltpu.make_async_copy(bufs[b3], dst(ci - 1), sout[b3]).wait()

                @pl.when(ci + _DEPTH - 1 < N)
                def _():
                    pltpu.make_async_copy(
                        src(ci + _DEPTH - 1), bufs[b3], sin[b3]).start()
            return carry

        lax.fori_loop(0, N // _DEPTH, quad_body, 0)
        last_b = (N - 1) % _DEPTH
        pltpu.make_async_copy(bufs[last_b], dst(N - 1), sout[last_b]).wait()

    return pl.pallas_call(
        body,
        in_specs=[
            pl.BlockSpec(memory_space=pltpu.MemorySpace.HBM),
            pl.BlockSpec(memory_space=pltpu.MemorySpace.HBM),
        ],
        out_specs=pl.BlockSpec(memory_space=pltpu.MemorySpace.HBM),
        out_shape=jax.ShapeDtypeStruct((B, S, D), x.dtype),
        scratch_shapes=(
            [pltpu.VMEM((S, D), jnp.float32)]
            + [pltpu.VMEM((_CH, D), jnp.float32) for _ in range(_DEPTH)]
            + [pltpu.SemaphoreType.DMA] * (1 + 2 * _DEPTH)
        ),
    )(x, pos_table)


# speedup vs baseline: 1.2015x; 1.0460x over previous
import jax
import jax.numpy as jnp
from jax import lax
from jax.experimental import pallas as pl
from jax.experimental.pallas import tpu as pltpu

_CH = 1024
_DEPTH = 4


def kernel(x, pos_table):
    B, S, D = x.shape
    cpb = S // _CH
    N = B * cpb

    def body(x_hbm, pos_hbm, out_hbm, pos_v, b0, b1, b2, b3,
             sp, si0, si1, si2, si3, so0, so1, so2, so3):
        bufs = (b0, b1, b2, b3)
        sin = (si0, si1, si2, si3)
        sout = (so0, so1, so2, so3)

        def src(ci):
            return x_hbm.at[ci // cpb, pl.ds((ci % cpb) * _CH, _CH)]

        def dst(ci):
            return out_hbm.at[ci // cpb, pl.ds((ci % cpb) * _CH, _CH)]

        pltpu.make_async_copy(pos_hbm, pos_v, sp).start()
        for j in range(_DEPTH - 1):
            pltpu.make_async_copy(src(j), bufs[j], sin[j]).start()
        pltpu.make_async_copy(pos_hbm, pos_v, sp).wait()

        def quad_body(g, carry):
            for b in range(_DEPTH):
                ci = _DEPTH * g + b
                pltpu.make_async_copy(src(ci), bufs[b], sin[b]).wait()
                kk = ci % cpb
                bufs[b][...] = bufs[b][...] + pos_v[pl.ds(kk * _CH, _CH), :]
                pltpu.make_async_copy(bufs[b], dst(ci), sout[b]).start()
                b3 = (b + _DEPTH - 1) % _DEPTH

                @pl.when(ci >= 1)
                def _():
                    pltpu.make_async_copy(bufs[b3], dst(ci - 1), sout[b3]).wait()

                @pl.when(ci + _DEPTH - 1 < N)
                def _():
                    pltpu.make_async_copy(
                        src(ci + _DEPTH - 1), bufs[b3], sin[b3]).start()
            return carry

        lax.fori_loop(0, N // _DEPTH, quad_body, 0)
        last_b = (N - 1) % _DEPTH
        pltpu.make_async_copy(bufs[last_b], dst(N - 1), sout[last_b]).wait()

    return pl.pallas_call(
        body,
        in_specs=[
            pl.BlockSpec(memory_space=pltpu.MemorySpace.HBM),
            pl.BlockSpec(memory_space=pltpu.MemorySpace.HBM),
        ],
        out_specs=pl.BlockSpec(memory_space=pltpu.MemorySpace.HBM),
        out_shape=jax.ShapeDtypeStruct((B, S, D), x.dtype),
        scratch_shapes=(
            [pltpu.VMEM((S, D), jnp.float32)]
            + [pltpu.VMEM((_CH, D), jnp.float32) for _ in range(_DEPTH)]
            + [pltpu.SemaphoreType.DMA] * (1 + 2 * _DEPTH)
        ),
    )(x, pos_table)
